# baseline (speedup 1.0000x reference)
import functools

import jax
import jax.numpy as jnp
from jax import lax
from jax.experimental import pallas as pl
from jax.experimental.pallas import tpu as pltpu
from jax.experimental.pallas import tpu_sc as plsc

f32 = jnp.float32
i32 = jnp.int32

_NC = 2
_NS = 16
_CH = 128
_BLK = 4
_ROWS = 640


def _sc_mesh():
  return plsc.VectorSubcoreMesh(
      core_axis_name="c", subcore_axis_name="s",
      num_cores=_NC, num_subcores=_NS)


@functools.cache
def _make_deg_kernel(npad, ep):
  stripe = npad // _NS
  et = ep // (_NC * _NS)
  nch = et // _CH

  @functools.partial(
      pl.kernel,
      out_type=jax.ShapeDtypeStruct((_NC * npad,), f32),
      mesh=_sc_mesh(),
      scratch_types=[
          pltpu.VMEM_SHARED((npad + 8,), f32),
          pltpu.VMEM((_CH,), i32),
          pltpu.VMEM((_CH,), f32),
          pltpu.VMEM((stripe,), f32),
      ],
  )
  def deg_kernel(dst_hbm, out_hbm, acc, idxv, onesv, zerov):
    cid = lax.axis_index("c")
    sid = lax.axis_index("s")

    @pl.loop(0, _CH // 16)
    def _(i):
      onesv[pl.ds(i * 16, 16)] = jnp.ones((16,), f32)

    @pl.loop(0, stripe // 16)
    def _(i):
      zerov[pl.ds(i * 16, 16)] = jnp.zeros((16,), f32)

    pltpu.sync_copy(zerov, acc.at[pl.ds(sid * stripe, stripe)])
    plsc.subcore_barrier()

    base = (cid * _NS + sid) * et

    @pl.loop(0, nch)
    def _(g):
      off = base + g * _CH
      pltpu.sync_copy(dst_hbm.at[pl.ds(off, _CH)], idxv)
      pltpu.sync_copy(
          onesv, acc.at[plsc.Indices(idxv, ignored_value=-1)], add=True)

    plsc.subcore_barrier()
    pltpu.sync_copy(acc.at[pl.ds(sid * stripe, stripe)], zerov)
    pltpu.sync_copy(zerov,
                    out_hbm.at[pl.ds(cid * npad + sid * stripe, stripe)])

  return deg_kernel


@functools.cache
def _make_agg_kernel(n, dh, ep, feature_split):
  stripe = n // _NS
  rows_total = ep // _CH
  if feature_split:
    tile_rows = rows_total // _NS
  else:
    tile_rows = rows_total // (_NC * _NS)
  nblk = tile_rows // _BLK

  @functools.partial(
      pl.kernel,
      out_type=(jax.ShapeDtypeStruct((n, dh), f32),) * 2,
      mesh=_sc_mesh(),
      scratch_types=[
          pltpu.VMEM_SHARED((n + 8, dh), f32),
          pltpu.VMEM((_BLK, _CH), i32),
          pltpu.VMEM((_BLK, _CH), i32),
          pltpu.VMEM((_CH, dh), f32),
          pltpu.VMEM((_CH, dh), f32),
          pltpu.SemaphoreType.DMA,
          pltpu.SemaphoreType.DMA,
      ],
  )
  def agg_kernel(in0_hbm, in1_hbm, src_hbm, dst_hbm, o0_hbm, o1_hbm,
                 acc, sblk, dblk, rows0, rows1, gsem0, gsem1):
    cid = lax.axis_index("c")
    sid = lax.axis_index("s")
    rows = (rows0, rows1)
    gsems = (gsem0, gsem1)

    for half in range(2):
      init_hbm = (in0_hbm, in1_hbm)[half]
      g_hbm = init_hbm if feature_split else in0_hbm
      o_hbm = (o0_hbm, o1_hbm)[half]
      if feature_split:
        rbase = sid * tile_rows
      else:
        rbase = half * (rows_total // _NC) + sid * tile_rows

      @pl.when(cid == half)
      def _():
        @pl.loop(0, stripe // _CH)
        def _(i):
          roff = sid * stripe + i * _CH
          pltpu.sync_copy(init_hbm.at[pl.ds(roff, _CH)], rows0)
          pltpu.sync_copy(rows0, acc.at[pl.ds(roff, _CH)])

        plsc.subcore_barrier()

        def start_gather(sl, b):
          return pltpu.async_copy(
              g_hbm.at[plsc.Indices(sblk.at[b])], rows[sl], gsems[sl])

        pltpu.sync_copy(src_hbm.at[pl.ds(rbase, _BLK)], sblk)
        pltpu.sync_copy(dst_hbm.at[pl.ds(rbase, _BLK)], dblk)
        start_gather(0, 0)

        @pl.loop(0, nblk)
        def _(i):
          for b in range(_BLK):
            sl = b % 2
            pltpu.make_async_copy(
                g_hbm.at[plsc.Indices(sblk.at[b])], rows[sl], gsems[sl]
            ).wait()
            if b < _BLK - 1:
              start_gather(sl ^ 1, b + 1)
            pltpu.sync_copy(
                rows[sl],
                acc.at[plsc.Indices(dblk.at[b], ignored_value=-1)],
                add=True)

          @pl.when(i < nblk - 1)
          def _():
            pltpu.sync_copy(
                src_hbm.at[pl.ds(rbase + (i + 1) * _BLK, _BLK)], sblk)
            pltpu.sync_copy(
                dst_hbm.at[pl.ds(rbase + (i + 1) * _BLK, _BLK)], dblk)
            start_gather(0, 0)

        plsc.subcore_barrier()

        @pl.loop(0, stripe // _CH)
        def _(i):
          roff = sid * stripe + i * _CH
          pltpu.sync_copy(acc.at[pl.ds(roff, _CH)], rows0)
          pltpu.sync_copy(rows0, o_hbm.at[pl.ds(roff, _CH)])

  return agg_kernel


def _silu(t):
  return t * (1.0 / (1.0 + jnp.exp(-t)))


@functools.cache
def _make_tc0(n, din):
  grid = n // _ROWS

  def body(parts_ref, x_ref, dinv_ref, s_ref):
    deg = jnp.sum(parts_ref[...], axis=1, keepdims=True) + 1.0
    dinv = lax.rsqrt(deg)
    dinv_ref[...] = dinv
    s_ref[...] = x_ref[...] * dinv

  return pl.pallas_call(
      body,
      grid=(grid,),
      in_specs=[
          pl.BlockSpec((_ROWS, _NC), lambda i: (i, 0)),
          pl.BlockSpec((_ROWS, din), lambda i: (i, 0)),
      ],
      out_specs=[
          pl.BlockSpec((_ROWS, 1), lambda i: (i, 0)),
          pl.BlockSpec((_ROWS, din), lambda i: (i, 0)),
      ],
      out_shape=[
          jax.ShapeDtypeStruct((n, 1), f32),
          jax.ShapeDtypeStruct((n, din), f32),
      ],
  )


@functools.cache
def _make_tc_mid(n, dh_in, dout, sum_partials):
  din = dh_in if sum_partials else 2 * dh_in
  dho = dout // 2
  grid = n // _ROWS

  def body(a0_ref, a1_ref, dinv_ref, w_ref, b_ref, ol_ref, or_ref):
    dinv = dinv_ref[...]
    w = w_ref[...]
    if sum_partials:
      t0 = (a0_ref[...] + a1_ref[...]) * dinv
      t = jnp.dot(t0, w, preferred_element_type=f32) + b_ref[...]
    else:
      tl = a0_ref[...] * dinv
      tr = a1_ref[...] * dinv
      t = (jnp.dot(tl, w[:dh_in], preferred_element_type=f32)
           + jnp.dot(tr, w[dh_in:], preferred_element_type=f32)
           + b_ref[...])
    s = _silu(t) * dinv
    ol_ref[...] = s[:, :dho]
    or_ref[...] = s[:, dho:]

  return pl.pallas_call(
      body,
      grid=(grid,),
      in_specs=[
          pl.BlockSpec((_ROWS, dh_in), lambda i: (i, 0)),
          pl.BlockSpec((_ROWS, dh_in), lambda i: (i, 0)),
          pl.BlockSpec((_ROWS, 1), lambda i: (i, 0)),
          pl.BlockSpec((din, dout), lambda i: (0, 0)),
          pl.BlockSpec((1, dout), lambda i: (0, 0)),
      ],
      out_specs=[
          pl.BlockSpec((_ROWS, dho), lambda i: (i, 0)),
          pl.BlockSpec((_ROWS, dho), lambda i: (i, 0)),
      ],
      out_shape=[
          jax.ShapeDtypeStruct((n, dho), f32),
          jax.ShapeDtypeStruct((n, dho), f32),
      ],
  )


@functools.cache
def _make_tc_fin(n, dh_in, dout):
  grid = n // _ROWS

  def body(al_ref, ar_ref, dinv_ref, wm_ref, bm_ref, wl_ref, bl_ref,
           mu_ref, lv_ref):
    dinv = dinv_ref[...]
    tl = al_ref[...] * dinv
    tr = ar_ref[...] * dinv
    wm = wm_ref[...]
    wl = wl_ref[...]
    mu_ref[...] = (jnp.dot(tl, wm[:dh_in], preferred_element_type=f32)
                   + jnp.dot(tr, wm[dh_in:], preferred_element_type=f32)
                   + bm_ref[...])
    lv_ref[...] = (jnp.dot(tl, wl[:dh_in], preferred_element_type=f32)
                   + jnp.dot(tr, wl[dh_in:], preferred_element_type=f32)
                   + bl_ref[...])

  din = 2 * dh_in
  return pl.pallas_call(
      body,
      grid=(grid,),
      in_specs=[
          pl.BlockSpec((_ROWS, dh_in), lambda i: (i, 0)),
          pl.BlockSpec((_ROWS, dh_in), lambda i: (i, 0)),
          pl.BlockSpec((_ROWS, 1), lambda i: (i, 0)),
          pl.BlockSpec((din, dout), lambda i: (0, 0)),
          pl.BlockSpec((1, dout), lambda i: (0, 0)),
          pl.BlockSpec((din, dout), lambda i: (0, 0)),
          pl.BlockSpec((1, dout), lambda i: (0, 0)),
      ],
      out_specs=[
          pl.BlockSpec((_ROWS, dout), lambda i: (i, 0)),
          pl.BlockSpec((_ROWS, dout), lambda i: (i, 0)),
      ],
      out_shape=[
          jax.ShapeDtypeStruct((n, dout), f32),
          jax.ShapeDtypeStruct((n, dout), f32),
      ],
  )


def kernel(x, edge_index, W0, b0, W1, b1, W2, b2, Wmu, bmu, Wlv, blv):
  n0, din = x.shape
  hid = W0.shape[1]
  e = edge_index.shape[1]

  n = ((n0 + _ROWS - 1) // _ROWS) * _ROWS
  if n != n0:
    x = jnp.pad(x, ((0, n - n0), (0, 0)))

  unit = _NC * _NS * _CH * _BLK
  ep = ((e + unit - 1) // unit) * unit
  pad = ep - e
  src = edge_index[0].astype(i32)
  dst = edge_index[1].astype(i32)
  if pad:
    srcp = jnp.concatenate([src, jnp.zeros((pad,), i32)])
    dstp = jnp.concatenate([dst, jnp.full((pad,), -1, i32)])
  else:
    srcp, dstp = src, dst
  src2 = srcp.reshape(ep // _CH, _CH)
  dst2 = dstp.reshape(ep // _CH, _CH)

  npad = n

  parts = _make_deg_kernel(npad, ep)(dstp)
  parts_t = parts.reshape(_NC, npad).T

  dinv, s0 = _make_tc0(n, din)(parts_t, x)

  zeros0 = jnp.zeros((n, din), f32)
  a0, a1 = _make_agg_kernel(n, din, ep, False)(s0, zeros0, src2, dst2)
  sl, sr = _make_tc_mid(n, din, hid, True)(a0, a1, dinv, W0, b0.reshape(1, -1))

  agg_hid = _make_agg_kernel(n, hid // 2, ep, True)
  tc_mid = _make_tc_mid(n, hid // 2, hid, False)
  al, ar = agg_hid(sl, sr, src2, dst2)
  sl, sr = tc_mid(al, ar, dinv, W1, b1.reshape(1, -1))
  al, ar = agg_hid(sl, sr, src2, dst2)
  sl, sr = tc_mid(al, ar, dinv, W2, b2.reshape(1, -1))
  al, ar = agg_hid(sl, sr, src2, dst2)

  mu, lv = _make_tc_fin(n, hid // 2, Wmu.shape[1])(
      al, ar, dinv, Wmu, bmu.reshape(1, -1), Wlv, blv.reshape(1, -1))
  return (mu[:n0], lv[:n0])


# speedup vs baseline: 1.6218x; 1.6218x over previous
import functools

import jax
import jax.numpy as jnp
from jax import lax
from jax.experimental import pallas as pl
from jax.experimental.pallas import tpu as pltpu
from jax.experimental.pallas import tpu_sc as plsc

f32 = jnp.float32
i32 = jnp.int32

_NC = 2
_NS = 16
_CH = 128
_BLK = 4
_ROWS = 640


def _sc_mesh():
  return plsc.VectorSubcoreMesh(
      core_axis_name="c", subcore_axis_name="s",
      num_cores=_NC, num_subcores=_NS)


@functools.cache
def _make_deg_kernel(npad, ep):
  stripe = npad // _NS
  et = ep // (_NC * _NS)
  nch = et // _CH

  @functools.partial(
      pl.kernel,
      out_type=jax.ShapeDtypeStruct((_NC * npad,), f32),
      mesh=_sc_mesh(),
      scratch_types=[
          pltpu.VMEM_SHARED((npad + 8,), f32),
          pltpu.VMEM((_CH,), i32),
          pltpu.VMEM((_CH,), f32),
          pltpu.VMEM((stripe,), f32),
      ],
  )
  def deg_kernel(dst_hbm, out_hbm, acc, idxv, onesv, zerov):
    cid = lax.axis_index("c")
    sid = lax.axis_index("s")

    @pl.loop(0, _CH // 16)
    def _(i):
      onesv[pl.ds(i * 16, 16)] = jnp.ones((16,), f32)

    @pl.loop(0, stripe // 16)
    def _(i):
      zerov[pl.ds(i * 16, 16)] = jnp.zeros((16,), f32)

    pltpu.sync_copy(zerov, acc.at[pl.ds(sid * stripe, stripe)])
    plsc.subcore_barrier()

    base = (cid * _NS + sid) * et

    @pl.loop(0, nch)
    def _(g):
      off = base + g * _CH
      pltpu.sync_copy(dst_hbm.at[pl.ds(off, _CH)], idxv)
      pltpu.sync_copy(
          onesv, acc.at[plsc.Indices(idxv, ignored_value=-1)], add=True)

    plsc.subcore_barrier()
    pltpu.sync_copy(acc.at[pl.ds(sid * stripe, stripe)], zerov)
    pltpu.sync_copy(zerov,
                    out_hbm.at[pl.ds(cid * npad + sid * stripe, stripe)])

  return deg_kernel


@functools.cache
def _make_agg_kernel(n, dh, ep, feature_split):
  stripe = n // _NS
  if feature_split:
    et = ep // _NS
  else:
    et = ep // (_NC * _NS)
  nch = et // _CH

  @functools.partial(
      pl.kernel,
      out_type=(jax.ShapeDtypeStruct((n, dh), f32),) * 2,
      mesh=_sc_mesh(),
      scratch_types=[
          pltpu.VMEM_SHARED((n, dh), f32),
          pltpu.VMEM((_CH,), i32),
          pltpu.VMEM((_CH,), i32),
          pltpu.VMEM((_CH, dh), f32),
          pltpu.SemaphoreType.DMA,
      ],
  )
  def agg_kernel(in0_hbm, in1_hbm, src_hbm, dst_hbm, o0_hbm, o1_hbm,
                 acc, siv, div, rows, sem):
    cid = lax.axis_index("c")
    sid = lax.axis_index("s")

    for half in range(2):
      init_hbm = (in0_hbm, in1_hbm)[half]
      g_hbm = init_hbm if feature_split else in0_hbm
      o_hbm = (o0_hbm, o1_hbm)[half]
      if feature_split:
        base = sid * et
      else:
        base = half * (ep // _NC) + sid * et

      @pl.when(cid == half)
      def _():
        @pl.loop(0, stripe // _CH)
        def _(i):
          roff = sid * stripe + i * _CH
          pltpu.sync_copy(init_hbm.at[pl.ds(roff, _CH)], rows)
          pltpu.sync_copy(rows, acc.at[pl.ds(roff, _CH)])

        plsc.subcore_barrier()

        @pl.loop(0, nch)
        def _(g):
          off = base + g * _CH
          pltpu.sync_copy(src_hbm.at[pl.ds(off, _CH)], siv)
          pltpu.sync_copy(dst_hbm.at[pl.ds(off, _CH)], div)
          pltpu.async_copy(
              g_hbm.at[plsc.Indices(siv, ignored_value=-1)], rows, sem
          ).wait()
          pltpu.sync_copy(
              rows, acc.at[plsc.Indices(div, ignored_value=-1)], add=True)

        plsc.subcore_barrier()

        @pl.loop(0, stripe // _CH)
        def _(i):
          roff = sid * stripe + i * _CH
          pltpu.sync_copy(acc.at[pl.ds(roff, _CH)], rows)
          pltpu.sync_copy(rows, o_hbm.at[pl.ds(roff, _CH)])

  return agg_kernel


def _silu(t):
  return t * (1.0 / (1.0 + jnp.exp(-t)))


@functools.cache
def _make_tc0(n, din):
  grid = n // _ROWS

  def body(parts_ref, x_ref, dinv_ref, s_ref):
    deg = jnp.sum(parts_ref[...], axis=1, keepdims=True) + 1.0
    dinv = lax.rsqrt(deg)
    dinv_ref[...] = dinv
    s_ref[...] = x_ref[...] * dinv

  return pl.pallas_call(
      body,
      grid=(grid,),
      in_specs=[
          pl.BlockSpec((_ROWS, _NC), lambda i: (i, 0)),
          pl.BlockSpec((_ROWS, din), lambda i: (i, 0)),
      ],
      out_specs=[
          pl.BlockSpec((_ROWS, 1), lambda i: (i, 0)),
          pl.BlockSpec((_ROWS, din), lambda i: (i, 0)),
      ],
      out_shape=[
          jax.ShapeDtypeStruct((n, 1), f32),
          jax.ShapeDtypeStruct((n, din), f32),
      ],
  )


@functools.cache
def _make_tc_mid(n, dh_in, dout, sum_partials):
  din = dh_in if sum_partials else 2 * dh_in
  dho = dout // 2
  grid = n // _ROWS

  def body(a0_ref, a1_ref, dinv_ref, w_ref, b_ref, ol_ref, or_ref):
    dinv = dinv_ref[...]
    w = w_ref[...]
    if sum_partials:
      t0 = (a0_ref[...] + a1_ref[...]) * dinv
      t = jnp.dot(t0, w, preferred_element_type=f32) + b_ref[...]
    else:
      tl = a0_ref[...] * dinv
      tr = a1_ref[...] * dinv
      t = (jnp.dot(tl, w[:dh_in], preferred_element_type=f32)
           + jnp.dot(tr, w[dh_in:], preferred_element_type=f32)
           + b_ref[...])
    s = _silu(t) * dinv
    ol_ref[...] = s[:, :dho]
    or_ref[...] = s[:, dho:]

  return pl.pallas_call(
      body,
      grid=(grid,),
      in_specs=[
          pl.BlockSpec((_ROWS, dh_in), lambda i: (i, 0)),
          pl.BlockSpec((_ROWS, dh_in), lambda i: (i, 0)),
          pl.BlockSpec((_ROWS, 1), lambda i: (i, 0)),
          pl.BlockSpec((din, dout), lambda i: (0, 0)),
          pl.BlockSpec((1, dout), lambda i: (0, 0)),
      ],
      out_specs=[
          pl.BlockSpec((_ROWS, dho), lambda i: (i, 0)),
          pl.BlockSpec((_ROWS, dho), lambda i: (i, 0)),
      ],
      out_shape=[
          jax.ShapeDtypeStruct((n, dho), f32),
          jax.ShapeDtypeStruct((n, dho), f32),
      ],
  )


@functools.cache
def _make_tc_fin(n, dh_in, dout):
  grid = n // _ROWS

  def body(al_ref, ar_ref, dinv_ref, wm_ref, bm_ref, wl_ref, bl_ref,
           mu_ref, lv_ref):
    dinv = dinv_ref[...]
    tl = al_ref[...] * dinv
    tr = ar_ref[...] * dinv
    wm = wm_ref[...]
    wl = wl_ref[...]
    mu_ref[...] = (jnp.dot(tl, wm[:dh_in], preferred_element_type=f32)
                   + jnp.dot(tr, wm[dh_in:], preferred_element_type=f32)
                   + bm_ref[...])
    lv_ref[...] = (jnp.dot(tl, wl[:dh_in], preferred_element_type=f32)
                   + jnp.dot(tr, wl[dh_in:], preferred_element_type=f32)
                   + bl_ref[...])

  din = 2 * dh_in
  return pl.pallas_call(
      body,
      grid=(grid,),
      in_specs=[
          pl.BlockSpec((_ROWS, dh_in), lambda i: (i, 0)),
          pl.BlockSpec((_ROWS, dh_in), lambda i: (i, 0)),
          pl.BlockSpec((_ROWS, 1), lambda i: (i, 0)),
          pl.BlockSpec((din, dout), lambda i: (0, 0)),
          pl.BlockSpec((1, dout), lambda i: (0, 0)),
          pl.BlockSpec((din, dout), lambda i: (0, 0)),
          pl.BlockSpec((1, dout), lambda i: (0, 0)),
      ],
      out_specs=[
          pl.BlockSpec((_ROWS, dout), lambda i: (i, 0)),
          pl.BlockSpec((_ROWS, dout), lambda i: (i, 0)),
      ],
      out_shape=[
          jax.ShapeDtypeStruct((n, dout), f32),
          jax.ShapeDtypeStruct((n, dout), f32),
      ],
  )


def kernel(x, edge_index, W0, b0, W1, b1, W2, b2, Wmu, bmu, Wlv, blv):
  n0, din = x.shape
  hid = W0.shape[1]
  e = edge_index.shape[1]

  n = ((n0 + _ROWS - 1) // _ROWS) * _ROWS
  if n != n0:
    x = jnp.pad(x, ((0, n - n0), (0, 0)))

  unit = _NC * _NS * _CH * _BLK
  ep = ((e + unit - 1) // unit) * unit
  pad = ep - e
  src = edge_index[0].astype(i32)
  dst = edge_index[1].astype(i32)
  if pad:
    fill = jnp.full((pad,), -1, i32)
    srcp = jnp.concatenate([src, fill])
    dstp = jnp.concatenate([dst, fill])
  else:
    srcp, dstp = src, dst

  npad = n

  parts = _make_deg_kernel(npad, ep)(dstp)
  parts_t = parts.reshape(_NC, npad).T

  dinv, s0 = _make_tc0(n, din)(parts_t, x)

  zeros0 = jnp.zeros((n, din), f32)
  a0, a1 = _make_agg_kernel(n, din, ep, False)(s0, zeros0, srcp, dstp)
  sl, sr = _make_tc_mid(n, din, hid, True)(a0, a1, dinv, W0, b0.reshape(1, -1))

  agg_hid = _make_agg_kernel(n, hid // 2, ep, True)
  tc_mid = _make_tc_mid(n, hid // 2, hid, False)
  al, ar = agg_hid(sl, sr, srcp, dstp)
  sl, sr = tc_mid(al, ar, dinv, W1, b1.reshape(1, -1))
  al, ar = agg_hid(sl, sr, srcp, dstp)
  sl, sr = tc_mid(al, ar, dinv, W2, b2.reshape(1, -1))
  al, ar = agg_hid(sl, sr, srcp, dstp)

  mu, lv = _make_tc_fin(n, hid // 2, Wmu.shape[1])(
      al, ar, dinv, Wmu, bmu.reshape(1, -1), Wlv, blv.reshape(1, -1))
  return (mu[:n0], lv[:n0])


# speedup vs baseline: 1.9835x; 1.2230x over previous
import functools

import jax
import jax.numpy as jnp
from jax import lax
from jax.experimental import pallas as pl
from jax.experimental.pallas import tpu as pltpu
from jax.experimental.pallas import tpu_sc as plsc

f32 = jnp.float32
i32 = jnp.int32

_NC = 2
_NS = 16
_CH = 128
_BLK = 4
_ROWS = 640


def _sc_mesh():
  return plsc.VectorSubcoreMesh(
      core_axis_name="c", subcore_axis_name="s",
      num_cores=_NC, num_subcores=_NS)


@functools.cache
def _make_deg_kernel(npad, ep):
  stripe = npad // _NS
  et = ep // (_NC * _NS)
  nch = et // _CH

  @functools.partial(
      pl.kernel,
      out_type=jax.ShapeDtypeStruct((_NC * npad,), f32),
      mesh=_sc_mesh(),
      scratch_types=[
          pltpu.VMEM_SHARED((npad + 8,), f32),
          pltpu.VMEM((_CH,), i32),
          pltpu.VMEM((_CH,), f32),
          pltpu.VMEM((stripe,), f32),
      ],
  )
  def deg_kernel(dst_hbm, out_hbm, acc, idxv, onesv, zerov):
    cid = lax.axis_index("c")
    sid = lax.axis_index("s")

    @pl.loop(0, _CH // 16)
    def _(i):
      onesv[pl.ds(i * 16, 16)] = jnp.ones((16,), f32)

    @pl.loop(0, stripe // 16)
    def _(i):
      zerov[pl.ds(i * 16, 16)] = jnp.zeros((16,), f32)

    pltpu.sync_copy(zerov, acc.at[pl.ds(sid * stripe, stripe)])
    plsc.subcore_barrier()

    base = (cid * _NS + sid) * et

    @pl.loop(0, nch)
    def _(g):
      off = base + g * _CH
      pltpu.sync_copy(dst_hbm.at[pl.ds(off, _CH)], idxv)
      pltpu.sync_copy(
          onesv, acc.at[plsc.Indices(idxv, ignored_value=-1)], add=True)

    plsc.subcore_barrier()
    pltpu.sync_copy(acc.at[pl.ds(sid * stripe, stripe)], zerov)
    pltpu.sync_copy(zerov,
                    out_hbm.at[pl.ds(cid * npad + sid * stripe, stripe)])

  return deg_kernel


@functools.cache
def _make_agg_kernel(n, dh, ep, feature_split):
  stripe = n // _NS
  if feature_split:
    et = ep // _NS
  else:
    et = ep // (_NC * _NS)
  nch = et // _CH

  @functools.partial(
      pl.kernel,
      out_type=(jax.ShapeDtypeStruct((n, dh), f32),) * 2,
      mesh=_sc_mesh(),
      scratch_types=[
          pltpu.VMEM_SHARED((n, dh), f32),
          pltpu.VMEM((_CH,), i32),
          pltpu.VMEM((_CH,), i32),
          pltpu.VMEM((_CH,), i32),
          pltpu.VMEM((_CH,), i32),
          pltpu.VMEM((_CH, dh), f32),
          pltpu.VMEM((_CH, dh), f32),
          pltpu.SemaphoreType.DMA,
          pltpu.SemaphoreType.DMA,
      ],
  )
  def agg_kernel(in0_hbm, in1_hbm, src_hbm, dst_hbm, o0_hbm, o1_hbm,
                 acc, siv0, siv1, div0, div1, rows0, rows1, sem0, sem1):
    cid = lax.axis_index("c")
    sid = lax.axis_index("s")
    sivs = (siv0, siv1)
    divs = (div0, div1)
    rows_b = (rows0, rows1)
    sems = (sem0, sem1)

    for half in range(2):
      init_hbm = (in0_hbm, in1_hbm)[half]
      g_hbm = init_hbm if feature_split else in0_hbm
      o_hbm = (o0_hbm, o1_hbm)[half]
      if feature_split:
        base = sid * et
      else:
        base = half * (ep // _NC) + sid * et

      @pl.when(cid == half)
      def _():
        @pl.loop(0, stripe // _CH)
        def _(i):
          roff = sid * stripe + i * _CH
          pltpu.sync_copy(init_hbm.at[pl.ds(roff, _CH)], rows0)
          pltpu.sync_copy(rows0, acc.at[pl.ds(roff, _CH)])

        plsc.subcore_barrier()

        def load_idx(sl, c):
          off = base + c * _CH
          pltpu.sync_copy(src_hbm.at[pl.ds(off, _CH)], sivs[sl])
          pltpu.sync_copy(dst_hbm.at[pl.ds(off, _CH)], divs[sl])

        def gather(sl):
          return pltpu.make_async_copy(
              g_hbm.at[plsc.Indices(sivs[sl], ignored_value=-1)],
              rows_b[sl], sems[sl])

        def scatter(sl):
          pltpu.sync_copy(
              rows_b[sl],
              acc.at[plsc.Indices(divs[sl], ignored_value=-1)], add=True)

        load_idx(0, 0)
        gather(0).start()

        @pl.loop(0, nch // 2)
        def _(i):
          gather(0).wait()
          load_idx(1, 2 * i + 1)
          gather(1).start()
          scatter(0)
          gather(1).wait()

          @pl.when(i < nch // 2 - 1)
          def _():
            load_idx(0, 2 * i + 2)
            gather(0).start()

          scatter(1)

        plsc.subcore_barrier()

        @pl.loop(0, stripe // _CH)
        def _(i):
          roff = sid * stripe + i * _CH
          pltpu.sync_copy(acc.at[pl.ds(roff, _CH)], rows0)
          pltpu.sync_copy(rows0, o_hbm.at[pl.ds(roff, _CH)])

  return agg_kernel


def _silu(t):
  return t * (1.0 / (1.0 + jnp.exp(-t)))


@functools.cache
def _make_tc0(n, din):
  grid = n // _ROWS

  def body(parts_ref, x_ref, dinv_ref, s_ref):
    deg = jnp.sum(parts_ref[...], axis=1, keepdims=True) + 1.0
    dinv = lax.rsqrt(deg)
    dinv_ref[...] = dinv
    s_ref[...] = x_ref[...] * dinv

  return pl.pallas_call(
      body,
      grid=(grid,),
      in_specs=[
          pl.BlockSpec((_ROWS, _NC), lambda i: (i, 0)),
          pl.BlockSpec((_ROWS, din), lambda i: (i, 0)),
      ],
      out_specs=[
          pl.BlockSpec((_ROWS, 1), lambda i: (i, 0)),
          pl.BlockSpec((_ROWS, din), lambda i: (i, 0)),
      ],
      out_shape=[
          jax.ShapeDtypeStruct((n, 1), f32),
          jax.ShapeDtypeStruct((n, din), f32),
      ],
  )


@functools.cache
def _make_tc_mid(n, dh_in, dout, sum_partials):
  din = dh_in if sum_partials else 2 * dh_in
  dho = dout // 2
  grid = n // _ROWS

  def body(a0_ref, a1_ref, dinv_ref, w_ref, b_ref, ol_ref, or_ref):
    dinv = dinv_ref[...]
    w = w_ref[...]
    if sum_partials:
      t0 = (a0_ref[...] + a1_ref[...]) * dinv
      t = jnp.dot(t0, w, preferred_element_type=f32) + b_ref[...]
    else:
      tl = a0_ref[...] * dinv
      tr = a1_ref[...] * dinv
      t = (jnp.dot(tl, w[:dh_in], preferred_element_type=f32)
           + jnp.dot(tr, w[dh_in:], preferred_element_type=f32)
           + b_ref[...])
    s = _silu(t) * dinv
    ol_ref[...] = s[:, :dho]
    or_ref[...] = s[:, dho:]

  return pl.pallas_call(
      body,
      grid=(grid,),
      in_specs=[
          pl.BlockSpec((_ROWS, dh_in), lambda i: (i, 0)),
          pl.BlockSpec((_ROWS, dh_in), lambda i: (i, 0)),
          pl.BlockSpec((_ROWS, 1), lambda i: (i, 0)),
          pl.BlockSpec((din, dout), lambda i: (0, 0)),
          pl.BlockSpec((1, dout), lambda i: (0, 0)),
      ],
      out_specs=[
          pl.BlockSpec((_ROWS, dho), lambda i: (i, 0)),
          pl.BlockSpec((_ROWS, dho), lambda i: (i, 0)),
      ],
      out_shape=[
          jax.ShapeDtypeStruct((n, dho), f32),
          jax.ShapeDtypeStruct((n, dho), f32),
      ],
  )


@functools.cache
def _make_tc_fin(n, dh_in, dout):
  grid = n // _ROWS

  def body(al_ref, ar_ref, dinv_ref, wm_ref, bm_ref, wl_ref, bl_ref,
           mu_ref, lv_ref):
    dinv = dinv_ref[...]
    tl = al_ref[...] * dinv
    tr = ar_ref[...] * dinv
    wm = wm_ref[...]
    wl = wl_ref[...]
    mu_ref[...] = (jnp.dot(tl, wm[:dh_in], preferred_element_type=f32)
                   + jnp.dot(tr, wm[dh_in:], preferred_element_type=f32)
                   + bm_ref[...])
    lv_ref[...] = (jnp.dot(tl, wl[:dh_in], preferred_element_type=f32)
                   + jnp.dot(tr, wl[dh_in:], preferred_element_type=f32)
                   + bl_ref[...])

  din = 2 * dh_in
  return pl.pallas_call(
      body,
      grid=(grid,),
      in_specs=[
          pl.BlockSpec((_ROWS, dh_in), lambda i: (i, 0)),
          pl.BlockSpec((_ROWS, dh_in), lambda i: (i, 0)),
          pl.BlockSpec((_ROWS, 1), lambda i: (i, 0)),
          pl.BlockSpec((din, dout), lambda i: (0, 0)),
          pl.BlockSpec((1, dout), lambda i: (0, 0)),
          pl.BlockSpec((din, dout), lambda i: (0, 0)),
          pl.BlockSpec((1, dout), lambda i: (0, 0)),
      ],
      out_specs=[
          pl.BlockSpec((_ROWS, dout), lambda i: (i, 0)),
          pl.BlockSpec((_ROWS, dout), lambda i: (i, 0)),
      ],
      out_shape=[
          jax.ShapeDtypeStruct((n, dout), f32),
          jax.ShapeDtypeStruct((n, dout), f32),
      ],
  )


def kernel(x, edge_index, W0, b0, W1, b1, W2, b2, Wmu, bmu, Wlv, blv):
  n0, din = x.shape
  hid = W0.shape[1]
  e = edge_index.shape[1]

  n = ((n0 + _ROWS - 1) // _ROWS) * _ROWS
  if n != n0:
    x = jnp.pad(x, ((0, n - n0), (0, 0)))

  unit = _NC * _NS * _CH * _BLK
  ep = ((e + unit - 1) // unit) * unit
  pad = ep - e
  src = edge_index[0].astype(i32)
  dst = edge_index[1].astype(i32)
  if pad:
    fill = jnp.full((pad,), -1, i32)
    srcp = jnp.concatenate([src, fill])
    dstp = jnp.concatenate([dst, fill])
  else:
    srcp, dstp = src, dst

  npad = n

  parts = _make_deg_kernel(npad, ep)(dstp)
  parts_t = parts.reshape(_NC, npad).T

  dinv, s0 = _make_tc0(n, din)(parts_t, x)

  zeros0 = jnp.zeros((n, din), f32)
  a0, a1 = _make_agg_kernel(n, din, ep, False)(s0, zeros0, srcp, dstp)
  sl, sr = _make_tc_mid(n, din, hid, True)(a0, a1, dinv, W0, b0.reshape(1, -1))

  agg_hid = _make_agg_kernel(n, hid // 2, ep, True)
  tc_mid = _make_tc_mid(n, hid // 2, hid, False)
  al, ar = agg_hid(sl, sr, srcp, dstp)
  sl, sr = tc_mid(al, ar, dinv, W1, b1.reshape(1, -1))
  al, ar = agg_hid(sl, sr, srcp, dstp)
  sl, sr = tc_mid(al, ar, dinv, W2, b2.reshape(1, -1))
  al, ar = agg_hid(sl, sr, srcp, dstp)

  mu, lv = _make_tc_fin(n, hid // 2, Wmu.shape[1])(
      al, ar, dinv, Wmu, bmu.reshape(1, -1), Wlv, blv.reshape(1, -1))
  return (mu[:n0], lv[:n0])


# speedup vs baseline: 2.5560x; 1.2887x over previous
import functools

import jax
import jax.numpy as jnp
from jax import lax
from jax.experimental import pallas as pl
from jax.experimental.pallas import tpu as pltpu
from jax.experimental.pallas import tpu_sc as plsc

f32 = jnp.float32
i32 = jnp.int32

_NC = 2
_NS = 16
_CH = 128
_BLK = 4
_ROWS = 640


def _sc_mesh():
  return plsc.VectorSubcoreMesh(
      core_axis_name="c", subcore_axis_name="s",
      num_cores=_NC, num_subcores=_NS)


@functools.cache
def _make_deg_kernel(npad, ep):
  stripe = npad // _NS
  et = ep // (_NC * _NS)
  nch = et // _CH

  @functools.partial(
      pl.kernel,
      out_type=jax.ShapeDtypeStruct((_NC * npad,), f32),
      mesh=_sc_mesh(),
      scratch_types=[
          pltpu.VMEM_SHARED((npad + 8,), f32),
          pltpu.VMEM((_CH,), i32),
          pltpu.VMEM((_CH,), f32),
          pltpu.VMEM((stripe,), f32),
      ],
  )
  def deg_kernel(dst_hbm, out_hbm, acc, idxv, onesv, zerov):
    cid = lax.axis_index("c")
    sid = lax.axis_index("s")

    @pl.loop(0, _CH // 16)
    def _(i):
      onesv[pl.ds(i * 16, 16)] = jnp.ones((16,), f32)

    @pl.loop(0, stripe // 16)
    def _(i):
      zerov[pl.ds(i * 16, 16)] = jnp.zeros((16,), f32)

    pltpu.sync_copy(zerov, acc.at[pl.ds(sid * stripe, stripe)])
    plsc.subcore_barrier()

    base = (cid * _NS + sid) * et

    @pl.loop(0, nch)
    def _(g):
      off = base + g * _CH
      pltpu.sync_copy(dst_hbm.at[pl.ds(off, _CH)], idxv)
      pltpu.sync_copy(
          onesv, acc.at[plsc.Indices(idxv, ignored_value=-1)], add=True)

    plsc.subcore_barrier()
    pltpu.sync_copy(acc.at[pl.ds(sid * stripe, stripe)], zerov)
    pltpu.sync_copy(zerov,
                    out_hbm.at[pl.ds(cid * npad + sid * stripe, stripe)])

  return deg_kernel


@functools.cache
def _make_agg_kernel(n, dh, ep, feature_split):
  stripe = n // _NS
  if feature_split:
    et = ep // _NS
  else:
    et = ep // (_NC * _NS)
  nch = et // _CH

  @functools.partial(
      pl.kernel,
      out_type=(jax.ShapeDtypeStruct((n, dh), f32),) * 2,
      mesh=_sc_mesh(),
      scratch_types=[
          pltpu.VMEM_SHARED((n, dh), f32),
          pltpu.VMEM((_CH,), i32),
          pltpu.VMEM((_CH,), i32),
          pltpu.VMEM((_CH,), i32),
          pltpu.VMEM((_CH,), i32),
          pltpu.VMEM((_CH, dh), f32),
          pltpu.VMEM((_CH, dh), f32),
          pltpu.SemaphoreType.DMA,
          pltpu.SemaphoreType.DMA,
          pltpu.SemaphoreType.DMA,
          pltpu.SemaphoreType.DMA,
      ],
  )
  def agg_kernel(in0_hbm, in1_hbm, src_hbm, dst_hbm, o0_hbm, o1_hbm,
                 acc, siv0, siv1, div0, div1, rows0, rows1, sem0, sem1,
                 ssem0, ssem1):
    cid = lax.axis_index("c")
    sid = lax.axis_index("s")
    sivs = (siv0, siv1)
    divs = (div0, div1)
    rows_b = (rows0, rows1)
    sems = (sem0, sem1)
    ssems = (ssem0, ssem1)

    for half in range(2):
      init_hbm = (in0_hbm, in1_hbm)[half]
      g_hbm = init_hbm if feature_split else in0_hbm
      o_hbm = (o0_hbm, o1_hbm)[half]
      if feature_split:
        base = sid * et
      else:
        base = half * (ep // _NC) + sid * et

      @pl.when(cid == half)
      def _():
        @pl.loop(0, stripe // _CH)
        def _(i):
          roff = sid * stripe + i * _CH
          pltpu.sync_copy(init_hbm.at[pl.ds(roff, _CH)], rows0)
          pltpu.sync_copy(rows0, acc.at[pl.ds(roff, _CH)])

        plsc.subcore_barrier()

        def load_idx(sl, c):
          off = base + c * _CH
          pltpu.sync_copy(src_hbm.at[pl.ds(off, _CH)], sivs[sl])
          pltpu.sync_copy(dst_hbm.at[pl.ds(off, _CH)], divs[sl])

        def gather(sl):
          return pltpu.make_async_copy(
              g_hbm.at[plsc.Indices(sivs[sl], ignored_value=-1)],
              rows_b[sl], sems[sl])

        def scatter(sl):
          return pltpu.make_async_copy(
              rows_b[sl],
              acc.at[plsc.Indices(divs[sl], ignored_value=-1)], ssems[sl])

        nb2 = nch // 2
        load_idx(0, 0)
        gather(0).start()

        @pl.loop(0, nb2)
        def _(i):
          @pl.when(i > 0)
          def _():
            scatter(1).wait()
          load_idx(1, 2 * i + 1)
          gather(0).wait()
          gather(1).start()
          scatter(0).start(add=True)

          @pl.when(i < nb2 - 1)
          def _():
            scatter(0).wait()
            load_idx(0, 2 * i + 2)
          gather(1).wait()

          @pl.when(i < nb2 - 1)
          def _():
            gather(0).start()
          scatter(1).start(add=True)

        scatter(0).wait()
        scatter(1).wait()
        plsc.subcore_barrier()

        @pl.loop(0, stripe // _CH)
        def _(i):
          roff = sid * stripe + i * _CH
          pltpu.sync_copy(acc.at[pl.ds(roff, _CH)], rows0)
          pltpu.sync_copy(rows0, o_hbm.at[pl.ds(roff, _CH)])

  return agg_kernel


def _silu(t):
  return t * (1.0 / (1.0 + jnp.exp(-t)))


@functools.cache
def _make_tc0(n, din):
  grid = n // _ROWS

  def body(parts_ref, x_ref, dinv_ref, s_ref):
    deg = jnp.sum(parts_ref[...], axis=1, keepdims=True) + 1.0
    dinv = lax.rsqrt(deg)
    dinv_ref[...] = dinv
    s_ref[...] = x_ref[...] * dinv

  return pl.pallas_call(
      body,
      grid=(grid,),
      in_specs=[
          pl.BlockSpec((_ROWS, _NC), lambda i: (i, 0)),
          pl.BlockSpec((_ROWS, din), lambda i: (i, 0)),
      ],
      out_specs=[
          pl.BlockSpec((_ROWS, 1), lambda i: (i, 0)),
          pl.BlockSpec((_ROWS, din), lambda i: (i, 0)),
      ],
      out_shape=[
          jax.ShapeDtypeStruct((n, 1), f32),
          jax.ShapeDtypeStruct((n, din), f32),
      ],
  )


@functools.cache
def _make_tc_mid(n, dh_in, dout, sum_partials):
  din = dh_in if sum_partials else 2 * dh_in
  dho = dout // 2
  grid = n // _ROWS

  def body(a0_ref, a1_ref, dinv_ref, w_ref, b_ref, ol_ref, or_ref):
    dinv = dinv_ref[...]
    w = w_ref[...]
    if sum_partials:
      t0 = (a0_ref[...] + a1_ref[...]) * dinv
      t = jnp.dot(t0, w, preferred_element_type=f32) + b_ref[...]
    else:
      tl = a0_ref[...] * dinv
      tr = a1_ref[...] * dinv
      t = (jnp.dot(tl, w[:dh_in], preferred_element_type=f32)
           + jnp.dot(tr, w[dh_in:], preferred_element_type=f32)
           + b_ref[...])
    s = _silu(t) * dinv
    ol_ref[...] = s[:, :dho]
    or_ref[...] = s[:, dho:]

  return pl.pallas_call(
      body,
      grid=(grid,),
      in_specs=[
          pl.BlockSpec((_ROWS, dh_in), lambda i: (i, 0)),
          pl.BlockSpec((_ROWS, dh_in), lambda i: (i, 0)),
          pl.BlockSpec((_ROWS, 1), lambda i: (i, 0)),
          pl.BlockSpec((din, dout), lambda i: (0, 0)),
          pl.BlockSpec((1, dout), lambda i: (0, 0)),
      ],
      out_specs=[
          pl.BlockSpec((_ROWS, dho), lambda i: (i, 0)),
          pl.BlockSpec((_ROWS, dho), lambda i: (i, 0)),
      ],
      out_shape=[
          jax.ShapeDtypeStruct((n, dho), f32),
          jax.ShapeDtypeStruct((n, dho), f32),
      ],
  )


@functools.cache
def _make_tc_fin(n, dh_in, dout):
  grid = n // _ROWS

  def body(al_ref, ar_ref, dinv_ref, wm_ref, bm_ref, wl_ref, bl_ref,
           mu_ref, lv_ref):
    dinv = dinv_ref[...]
    tl = al_ref[...] * dinv
    tr = ar_ref[...] * dinv
    wm = wm_ref[...]
    wl = wl_ref[...]
    mu_ref[...] = (jnp.dot(tl, wm[:dh_in], preferred_element_type=f32)
                   + jnp.dot(tr, wm[dh_in:], preferred_element_type=f32)
                   + bm_ref[...])
    lv_ref[...] = (jnp.dot(tl, wl[:dh_in], preferred_element_type=f32)
                   + jnp.dot(tr, wl[dh_in:], preferred_element_type=f32)
                   + bl_ref[...])

  din = 2 * dh_in
  return pl.pallas_call(
      body,
      grid=(grid,),
      in_specs=[
          pl.BlockSpec((_ROWS, dh_in), lambda i: (i, 0)),
          pl.BlockSpec((_ROWS, dh_in), lambda i: (i, 0)),
          pl.BlockSpec((_ROWS, 1), lambda i: (i, 0)),
          pl.BlockSpec((din, dout), lambda i: (0, 0)),
          pl.BlockSpec((1, dout), lambda i: (0, 0)),
          pl.BlockSpec((din, dout), lambda i: (0, 0)),
          pl.BlockSpec((1, dout), lambda i: (0, 0)),
      ],
      out_specs=[
          pl.BlockSpec((_ROWS, dout), lambda i: (i, 0)),
          pl.BlockSpec((_ROWS, dout), lambda i: (i, 0)),
      ],
      out_shape=[
          jax.ShapeDtypeStruct((n, dout), f32),
          jax.ShapeDtypeStruct((n, dout), f32),
      ],
  )


def kernel(x, edge_index, W0, b0, W1, b1, W2, b2, Wmu, bmu, Wlv, blv):
  n0, din = x.shape
  hid = W0.shape[1]
  e = edge_index.shape[1]

  n = ((n0 + _ROWS - 1) // _ROWS) * _ROWS
  if n != n0:
    x = jnp.pad(x, ((0, n - n0), (0, 0)))

  unit = _NC * _NS * _CH * _BLK
  ep = ((e + unit - 1) // unit) * unit
  pad = ep - e
  src = edge_index[0].astype(i32)
  dst = edge_index[1].astype(i32)
  if pad:
    fill = jnp.full((pad,), -1, i32)
    srcp = jnp.concatenate([src, fill])
    dstp = jnp.concatenate([dst, fill])
  else:
    srcp, dstp = src, dst

  npad = n

  parts = _make_deg_kernel(npad, ep)(dstp)
  parts_t = parts.reshape(_NC, npad).T

  dinv, s0 = _make_tc0(n, din)(parts_t, x)

  zeros0 = jnp.zeros((n, din), f32)
  a0, a1 = _make_agg_kernel(n, din, ep, False)(s0, zeros0, srcp, dstp)
  sl, sr = _make_tc_mid(n, din, hid, True)(a0, a1, dinv, W0, b0.reshape(1, -1))

  agg_hid = _make_agg_kernel(n, hid // 2, ep, True)
  tc_mid = _make_tc_mid(n, hid // 2, hid, False)
  al, ar = agg_hid(sl, sr, srcp, dstp)
  sl, sr = tc_mid(al, ar, dinv, W1, b1.reshape(1, -1))
  al, ar = agg_hid(sl, sr, srcp, dstp)
  sl, sr = tc_mid(al, ar, dinv, W2, b2.reshape(1, -1))
  al, ar = agg_hid(sl, sr, srcp, dstp)

  mu, lv = _make_tc_fin(n, hid // 2, Wmu.shape[1])(
      al, ar, dinv, Wmu, bmu.reshape(1, -1), Wlv, blv.reshape(1, -1))
  return (mu[:n0], lv[:n0])
